# Initial kernel scaffold; baseline (speedup 1.0000x reference)
import jax
import jax.numpy as jnp
from jax.experimental import pallas as pl


def kernel(x, edge_index, W_self, W_neigh, b):
    raise NotImplementedError("write your pallas kernel here")


# speedup vs baseline: 7.6446x; 7.6446x over previous
import functools

import jax
import jax.numpy as jnp
from jax import lax
from jax.experimental import pallas as pl
from jax.experimental.pallas import tpu as pltpu
from jax.experimental.pallas import tpu_sc as plsc

N_NODES = 10000
N_EDGES = 320000
D = 128
DP = 144
NC = 2
NS = 16
NW = NC * NS
CHUNK = 80
EDGES_PER_TILE = N_EDGES // NW
CHUNKS_PER_TILE = EDGES_PER_TILE // CHUNK
N_PAD = 10240
ROWS_PER_TILE = N_PAD // NS


def _sc_aggregate(xpad, src2, dst2):
    mesh = plsc.VectorSubcoreMesh(core_axis_name="c", subcore_axis_name="s")

    @functools.partial(
        pl.kernel,
        out_type=jax.ShapeDtypeStruct((NC * N_PAD, DP), jnp.float32),
        mesh=mesh,
        scratch_types=[
            pltpu.VMEM((CHUNKS_PER_TILE, CHUNK), jnp.int32),
            pltpu.VMEM((CHUNKS_PER_TILE, CHUNK), jnp.int32),
            pltpu.VMEM((CHUNK, DP), jnp.float32),
            pltpu.VMEM_SHARED((N_PAD, DP), jnp.float32),
            pltpu.SemaphoreType.DMA,
        ],
        compiler_params=pltpu.CompilerParams(use_tc_tiling_on_sc=False),
    )
    def body(xpad_hbm, src_hbm, dst_hbm, out_hbm, src_v, dst_v, rows_v, acc_sh, sem):
        c = lax.axis_index("c")
        s = lax.axis_index("s")
        w = c * NS + s

        pltpu.sync_copy(src_hbm.at[w], src_v)
        pltpu.sync_copy(dst_hbm.at[w], dst_v)

        zero16 = jnp.zeros((16,), jnp.float32)

        def zero_row(i, carry):
            for k in range(DP // 16):
                rows_v[i, pl.ds(k * 16, 16)] = zero16
            return carry

        lax.fori_loop(0, CHUNK, zero_row, 0)

        base = s * ROWS_PER_TILE
        for k in range(ROWS_PER_TILE // CHUNK):
            pltpu.sync_copy(rows_v, acc_sh.at[pl.ds(base + k * CHUNK, CHUNK)])
        plsc.subcore_barrier()

        def edge_chunk(j, carry):
            pltpu.async_copy(xpad_hbm.at[src_v.at[j]], rows_v, sem).wait()
            pltpu.sync_copy(rows_v, acc_sh.at[dst_v.at[j]], add=True)
            return carry

        lax.fori_loop(0, CHUNKS_PER_TILE, edge_chunk, 0)
        plsc.subcore_barrier()

        pltpu.sync_copy(acc_sh.at[pl.ds(base, ROWS_PER_TILE)],
                        out_hbm.at[pl.ds(c * N_PAD + base, ROWS_PER_TILE)])

    return body(xpad, src2, dst2)


def _tc_combine(x, parts, W_self, W_neigh, b):
    R = 1000

    def body(x_ref, p_ref, ws_ref, wn_ref, b_ref, o_ref):
        p = p_ref[...]
        agg = p[0, :, :D] + p[1, :, :D]
        cnt = p[0, :, D:D + 1] + p[1, :, D:D + 1]
        mean = agg / jnp.maximum(cnt, 1.0)
        o_ref[...] = (
            jnp.dot(x_ref[...], ws_ref[...], preferred_element_type=jnp.float32)
            + jnp.dot(mean, wn_ref[...], preferred_element_type=jnp.float32)
            + b_ref[...]
        )

    return pl.pallas_call(
        body,
        grid=(N_NODES // R,),
        in_specs=[
            pl.BlockSpec((R, D), lambda i: (i, 0)),
            pl.BlockSpec((NC, R, DP), lambda i: (0, i, 0)),
            pl.BlockSpec((D, D), lambda i: (0, 0)),
            pl.BlockSpec((D, D), lambda i: (0, 0)),
            pl.BlockSpec((1, D), lambda i: (0, 0)),
        ],
        out_specs=pl.BlockSpec((R, D), lambda i: (i, 0)),
        out_shape=jax.ShapeDtypeStruct((N_NODES, D), jnp.float32),
    )(x, parts, W_self, W_neigh, b.reshape(1, D))


def kernel(x, edge_index, W_self, W_neigh, b):
    ones_col = jnp.ones((N_NODES, 1), jnp.float32)
    pad = jnp.zeros((N_NODES, DP - D - 1), jnp.float32)
    xpad = jnp.concatenate([x, ones_col, pad], axis=1)
    src2 = edge_index[0].reshape(NW, CHUNKS_PER_TILE, CHUNK)
    dst2 = edge_index[1].reshape(NW, CHUNKS_PER_TILE, CHUNK)
    parts = _sc_aggregate(xpad, src2, dst2).reshape(NC, N_PAD, DP)
    return _tc_combine(x, parts, W_self, W_neigh, b)


# speedup vs baseline: 9.9060x; 1.2958x over previous
import functools

import jax
import jax.numpy as jnp
from jax import lax
from jax.experimental import pallas as pl
from jax.experimental.pallas import tpu as pltpu
from jax.experimental.pallas import tpu_sc as plsc

N_NODES = 10000
N_EDGES = 320000
D = 128
DP = 144
NC = 2
NS = 16
NW = NC * NS
CHUNK = 125
EDGES_PER_TILE = N_EDGES // NW
CHUNKS_PER_TILE = EDGES_PER_TILE // CHUNK
N_PAD = 10240
ROWS_PER_TILE = N_PAD // NS


def _sc_aggregate(xpad, src2, dst2):
    mesh = plsc.VectorSubcoreMesh(core_axis_name="c", subcore_axis_name="s")

    @functools.partial(
        pl.kernel,
        out_type=jax.ShapeDtypeStruct((NC * N_PAD, DP), jnp.float32),
        mesh=mesh,
        scratch_types=[
            pltpu.VMEM((CHUNK,), jnp.int32),
            pltpu.VMEM((CHUNK,), jnp.int32),
            pltpu.VMEM((CHUNK,), jnp.int32),
            pltpu.VMEM((CHUNK,), jnp.int32),
            pltpu.VMEM((CHUNK, DP), jnp.float32),
            pltpu.VMEM((CHUNK, DP), jnp.float32),
            pltpu.VMEM_SHARED((N_PAD, DP), jnp.float32),
            pltpu.SemaphoreType.DMA,
            pltpu.SemaphoreType.DMA,
            pltpu.SemaphoreType.DMA,
            pltpu.SemaphoreType.DMA,
        ],
        compiler_params=pltpu.CompilerParams(use_tc_tiling_on_sc=False),
    )
    def body(xpad_hbm, src_hbm, dst_hbm, out_hbm,
             src0, dst0, src1, dst1, rows0, rows1, acc_sh,
             sem_i0, sem_i1, sem_r0, sem_r1):
        c = lax.axis_index("c")
        s = lax.axis_index("s")
        w = c * NS + s
        idx = src_hbm.at[w]
        idy = dst_hbm.at[w]

        zero16 = jnp.zeros((16,), jnp.float32)

        def zero_row(i, carry):
            for k in range(DP // 16):
                rows0[i, pl.ds(k * 16, 16)] = zero16
            return carry

        lax.fori_loop(0, CHUNK, zero_row, 0)

        base = s * ROWS_PER_TILE
        full, rem = divmod(ROWS_PER_TILE, CHUNK)
        for k in range(full):
            pltpu.sync_copy(rows0, acc_sh.at[pl.ds(base + k * CHUNK, CHUNK)])
        if rem:
            pltpu.sync_copy(rows0.at[pl.ds(0, rem)],
                            acc_sh.at[pl.ds(base + full * CHUNK, rem)])
        plsc.subcore_barrier()

        NCH = CHUNKS_PER_TILE
        pltpu.sync_copy(idx.at[0], src0)
        pltpu.sync_copy(idy.at[0], dst0)
        pltpu.async_copy(idx.at[1], src1, sem_i1)
        pltpu.async_copy(idy.at[1], dst1, sem_i1)
        pltpu.async_copy(xpad_hbm.at[src0], rows0, sem_r0)

        def pair(g, carry):
            j0 = 2 * g
            pltpu.make_async_copy(xpad_hbm.at[src0], rows0, sem_r0).wait()
            pltpu.make_async_copy(idx.at[j0], src1, sem_i1).wait()
            pltpu.make_async_copy(idy.at[j0], dst1, sem_i1).wait()
            pltpu.async_copy(xpad_hbm.at[src1], rows1, sem_r1)
            pltpu.sync_copy(rows0, acc_sh.at[dst0], add=True)

            @pl.when(g < NCH // 2 - 1)
            def _():
                pltpu.async_copy(idx.at[j0 + 2], src0, sem_i0)
                pltpu.async_copy(idy.at[j0 + 2], dst0, sem_i0)

            pltpu.make_async_copy(xpad_hbm.at[src1], rows1, sem_r1).wait()

            @pl.when(g < NCH // 2 - 1)
            def _():
                pltpu.make_async_copy(idx.at[j0], src0, sem_i0).wait()
                pltpu.make_async_copy(idy.at[j0], dst0, sem_i0).wait()
                pltpu.async_copy(xpad_hbm.at[src0], rows0, sem_r0)

            pltpu.sync_copy(rows1, acc_sh.at[dst1], add=True)

            @pl.when(g < NCH // 2 - 1)
            def _():
                pltpu.async_copy(idx.at[j0 + 3], src1, sem_i1)
                pltpu.async_copy(idy.at[j0 + 3], dst1, sem_i1)

            return carry

        lax.fori_loop(0, NCH // 2, pair, 0)
        plsc.subcore_barrier()

        pltpu.sync_copy(acc_sh.at[pl.ds(base, ROWS_PER_TILE)],
                        out_hbm.at[pl.ds(c * N_PAD + base, ROWS_PER_TILE)])

    return body(xpad, src2, dst2)


def _tc_combine(x, parts, W_self, W_neigh, b):
    R = 1000

    def body(x_ref, p_ref, ws_ref, wn_ref, b_ref, o_ref):
        p = p_ref[...]
        agg = p[0, :, :D] + p[1, :, :D]
        cnt = p[0, :, D:D + 1] + p[1, :, D:D + 1]
        mean = agg / jnp.maximum(cnt, 1.0)
        o_ref[...] = (
            jnp.dot(x_ref[...], ws_ref[...], preferred_element_type=jnp.float32)
            + jnp.dot(mean, wn_ref[...], preferred_element_type=jnp.float32)
            + b_ref[...]
        )

    return pl.pallas_call(
        body,
        grid=(N_NODES // R,),
        in_specs=[
            pl.BlockSpec((R, D), lambda i: (i, 0)),
            pl.BlockSpec((NC, R, DP), lambda i: (0, i, 0)),
            pl.BlockSpec((D, D), lambda i: (0, 0)),
            pl.BlockSpec((D, D), lambda i: (0, 0)),
            pl.BlockSpec((1, D), lambda i: (0, 0)),
        ],
        out_specs=pl.BlockSpec((R, D), lambda i: (i, 0)),
        out_shape=jax.ShapeDtypeStruct((N_NODES, D), jnp.float32),
    )(x, parts, W_self, W_neigh, b.reshape(1, D))


def kernel(x, edge_index, W_self, W_neigh, b):
    ones_col = jnp.ones((N_NODES, 1), jnp.float32)
    pad = jnp.zeros((N_NODES, DP - D - 1), jnp.float32)
    xpad = jnp.concatenate([x, ones_col, pad], axis=1)
    src2 = edge_index[0].reshape(NW, CHUNKS_PER_TILE, CHUNK)
    dst2 = edge_index[1].reshape(NW, CHUNKS_PER_TILE, CHUNK)
    parts = _sc_aggregate(xpad, src2, dst2).reshape(NC, N_PAD, DP)
    return _tc_combine(x, parts, W_self, W_neigh, b)
